# baseline (device time: 17278 ns/iter reference)
import jax
import jax.numpy as jnp
from jax import lax
from jax.experimental import pallas as pl
from jax.experimental.pallas import tpu as pltpu

VOCAB_SLAB = 1024


def kernel(ids, E):
    T = ids.shape[0]
    V_LOC, D = E.shape

    def body(ids_ref, e_ref, out_ref, send_buf, recv_buf, send_sem, recv_sem):
        my_x = lax.axis_index("x")
        my_y = lax.axis_index("y")
        my_z = lax.axis_index("z")
        partner = (1 - my_x, my_y, my_z)

        barrier_sem = pltpu.get_barrier_semaphore()
        pl.semaphore_signal(
            barrier_sem, inc=1,
            device_id=partner, device_id_type=pl.DeviceIdType.MESH,
        )
        pl.semaphore_wait(barrier_sem, 1)

        ids_col = (ids_ref[:] - my_x * V_LOC).reshape(T, 1)
        acc = jnp.zeros((T, D), jnp.float32)
        for k in range(V_LOC // VOCAB_SLAB):
            iota = lax.broadcasted_iota(jnp.int32, (T, VOCAB_SLAB), 1)
            oh = (iota + k * VOCAB_SLAB == ids_col).astype(jnp.bfloat16)
            e_slab = e_ref[k * VOCAB_SLAB:(k + 1) * VOCAB_SLAB, :].astype(
                jnp.bfloat16
            )
            acc = acc + lax.dot_general(
                oh, e_slab, (((1,), (0,)), ((), ())),
                preferred_element_type=jnp.float32,
            )
        send_buf[:, :] = acc.astype(jnp.bfloat16)

        rdma = pltpu.make_async_remote_copy(
            src_ref=send_buf,
            dst_ref=recv_buf,
            send_sem=send_sem,
            recv_sem=recv_sem,
            device_id=partner,
            device_id_type=pl.DeviceIdType.MESH,
        )
        rdma.start()
        rdma.wait()

        out_ref[:, :] = acc + recv_buf[:, :].astype(jnp.float32)

    return pl.pallas_call(
        body,
        out_shape=jax.ShapeDtypeStruct((T, D), jnp.float32),
        in_specs=[
            pl.BlockSpec(memory_space=pltpu.VMEM),
            pl.BlockSpec(memory_space=pltpu.VMEM),
        ],
        out_specs=pl.BlockSpec(memory_space=pltpu.VMEM),
        scratch_shapes=[
            pltpu.VMEM((T, D), jnp.bfloat16),
            pltpu.VMEM((T, D), jnp.bfloat16),
            pltpu.SemaphoreType.DMA,
            pltpu.SemaphoreType.DMA,
        ],
        compiler_params=pltpu.CompilerParams(collective_id=0),
    )(ids, E)
